# baseline (speedup 1.0000x reference)
import functools

import jax
import jax.numpy as jnp
from jax import lax
from jax.experimental import pallas as pl
from jax.experimental.pallas import tpu as pltpu
from jax.experimental.pallas import tpu_sc as plsc

N, K, D, H = 10000, 32, 128, 512
BN = 200
GRID = N // BN
NE = N * K
ECHUNK = 128
NW = 32
C = 2
NE_C = NE // C
NC_CALL = -(-NE_C // ECHUNK // (2 * NW)) * 2 * NW
PW = NC_CALL // NW
NE_CALL = NC_CALL * ECHUNK
GC = GRID // C


def _ln(x, w, b):
    m = jnp.mean(x, axis=-1, keepdims=True)
    v = jnp.mean(jnp.square(x - m), axis=-1, keepdims=True)
    return (x - m) * lax.rsqrt(v + 1e-5) * w + b


def _gelu(x):
    return 0.5 * x * (1.0 + lax.erf(x * 0.7071067811865476))


def _prep_body(h_ref, at_ref, ct_ref, b0_ref, p_ref, q_ref):
    h = h_ref[...]
    p_ref[...] = jnp.dot(h, at_ref[...], preferred_element_type=jnp.float32) + b0_ref[...]
    q_ref[...] = jnp.dot(h, ct_ref[...], preferred_element_type=jnp.float32
                         ).astype(jnp.bfloat16)


def _prep(h, At, Ct, b0):
    return pl.pallas_call(
        _prep_body,
        out_shape=(
            jax.ShapeDtypeStruct((N, D), jnp.float32),
            jax.ShapeDtypeStruct((N, D), jnp.bfloat16),
        ),
    )(h, At, Ct, b0.reshape(1, D))


def _gather_body(table_hbm, idx_hbm, out_hbm, idx_v, rows0, rows1, sem0, sem1):
    wid = lax.axis_index("s") * 2 + lax.axis_index("c")
    base = wid * PW

    pltpu.sync_copy(idx_hbm.at[pl.ds(base, PW)], idx_v)

    def start(c, rows, sem):
        pltpu.async_copy(table_hbm.at[idx_v.at[c]], rows, sem)

    def wait(rows, sem):
        pltpu.make_async_copy(out_hbm.at[pl.ds(0, ECHUNK)], rows, sem).wait()

    def writeback(c, rows):
        pltpu.sync_copy(rows, out_hbm.at[pl.ds((base + c) * ECHUNK, ECHUNK)])

    start(0, rows0, sem0)

    def body(t, carry):
        c0 = 2 * t
        start(c0 + 1, rows1, sem1)
        wait(rows0, sem0)
        writeback(c0, rows0)

        @pl.when(c0 + 2 < PW)
        def _():
            start(c0 + 2, rows0, sem0)

        wait(rows1, sem1)
        writeback(c0 + 1, rows1)
        return carry

    lax.fori_loop(0, PW // 2, body, 0)


@functools.lru_cache(maxsize=None)
def _make_gather():
    return pl.kernel(
        _gather_body,
        out_type=jax.ShapeDtypeStruct((NE_CALL, D // 2), jnp.int32),
        mesh=plsc.VectorSubcoreMesh(core_axis_name="c", subcore_axis_name="s"),
        scratch_types=[
            pltpu.VMEM((PW, ECHUNK), jnp.int32),
            pltpu.VMEM((ECHUNK, D // 2), jnp.int32),
            pltpu.VMEM((ECHUNK, D // 2), jnp.int32),
            pltpu.SemaphoreType.DMA,
            pltpu.SemaphoreType.DMA,
        ],
        compiler_params=pltpu.CompilerParams(use_tc_tiling_on_sc=False),
    )


def _gather(table, idx):
    return _make_gather()(table, idx)


def _unpack_qj(w):
    lo = lax.bitcast_convert_type(jnp.left_shift(w, 16), jnp.float32)
    hi = lax.bitcast_convert_type(
        jnp.bitwise_and(w, jnp.int32(-65536)), jnp.float32)
    return jnp.concatenate([lo, hi], axis=-1)


def _phase1_body(e_ref, qj_ref, p_ref, h_ref, mask_ref,
                 bt_ref, w1t_ref, b1_ref, w2t_ref, b2_ref,
                 ln1w_ref, ln1b_ref,
                 dw0t_ref, db0_ref, dw1t_ref, db1_ref,
                 ln2w_ref, ln2b_ref, a2t_ref, c2t_ref, eub0_ref,
                 out_ref, p2_ref, q2_ref):
    e = e_ref[...].reshape(BN * K, D)
    qj = _unpack_qj(qj_ref[...])
    x = jnp.dot(e, bt_ref[...], preferred_element_type=jnp.float32) + qj
    p = jnp.broadcast_to(p_ref[...][:, None, :], (BN, K, D)).reshape(BN * K, D)
    x = _gelu(x + p)
    x = _gelu(jnp.dot(x, w1t_ref[...], preferred_element_type=jnp.float32) + b1_ref[...])
    m = jnp.dot(x, w2t_ref[...], preferred_element_type=jnp.float32) + b2_ref[...]
    agg = jnp.sum(m.reshape(BN, K, D), axis=1) * (1.0 / 30.0)
    h1 = _ln(h_ref[...] + agg, ln1w_ref[...], ln1b_ref[...])
    d = _gelu(jnp.dot(h1, dw0t_ref[...], preferred_element_type=jnp.float32) + db0_ref[...])
    h2 = h1 + jnp.dot(d, dw1t_ref[...], preferred_element_type=jnp.float32) + db1_ref[...]
    h2 = _ln(h2, ln2w_ref[...], ln2b_ref[...])
    h2 = h2 * mask_ref[...]
    out_ref[...] = h2
    p2_ref[...] = jnp.dot(h2, a2t_ref[...], preferred_element_type=jnp.float32) + eub0_ref[...]
    q2_ref[...] = jnp.dot(h2, c2t_ref[...], preferred_element_type=jnp.float32
                          ).astype(jnp.bfloat16)


def _phase1(c, e, qj, p, h, mask2d, Bt, W1t, b1, W2t, b2, ln1w, ln1b,
            dW0t, db0, dW1t, db1, ln2w, ln2b, A2t, C2t, eu_b0):
    off = c * GC
    full = lambda shape: pl.BlockSpec(shape, lambda i: (0,) * len(shape))
    nd_off = pl.BlockSpec((BN, D), lambda i: (i + off, 0))
    nd_loc = pl.BlockSpec((BN, D), lambda i: (i, 0))
    return pl.pallas_call(
        _phase1_body,
        grid=(GC,),
        in_specs=[
            pl.BlockSpec((BN, K, D), lambda i: (i + off, 0, 0)),
            pl.BlockSpec((BN * K, D // 2), lambda i: (i, 0)),
            nd_off,
            nd_off,
            pl.BlockSpec((BN, 1), lambda i: (i + off, 0)),
            full((D, D)), full((D, D)), full((1, D)), full((D, D)), full((1, D)),
            full((1, D)), full((1, D)),
            full((D, H)), full((1, H)), full((H, D)), full((1, D)),
            full((1, D)), full((1, D)),
            full((D, D)), full((D, D)), full((1, D)),
        ],
        out_specs=(nd_loc, nd_loc, nd_loc),
        out_shape=(
            jax.ShapeDtypeStruct((N // C, D), jnp.float32),
            jax.ShapeDtypeStruct((N // C, D), jnp.float32),
            jax.ShapeDtypeStruct((N // C, D), jnp.bfloat16),
        ),
    )(e, qj, p, h, mask2d, Bt, W1t, b1.reshape(1, D), W2t, b2.reshape(1, D),
      ln1w.reshape(1, D), ln1b.reshape(1, D),
      dW0t, db0.reshape(1, H), dW1t, db1.reshape(1, D),
      ln2w.reshape(1, D), ln2b.reshape(1, D), A2t, C2t, eu_b0.reshape(1, D))


def _phase2_body(acc_ref, e_ref, qj_ref, p_ref,
                 bt_ref, w1t_ref, b1_ref, w2t_ref, b2_ref,
                 ln3w_ref, ln3b_ref, out_ref):
    del acc_ref
    e = e_ref[...].reshape(BN * K, D)
    qj = _unpack_qj(qj_ref[...])
    x = jnp.dot(e, bt_ref[...], preferred_element_type=jnp.float32) + qj
    p = jnp.broadcast_to(p_ref[...][:, None, :], (BN, K, D)).reshape(BN * K, D)
    x = _gelu(x + p)
    x = _gelu(jnp.dot(x, w1t_ref[...], preferred_element_type=jnp.float32) + b1_ref[...])
    m = jnp.dot(x, w2t_ref[...], preferred_element_type=jnp.float32) + b2_ref[...]
    out = _ln(e + m, ln3w_ref[...], ln3b_ref[...])
    out_ref[...] = out.reshape(BN, K, D)


def _phase2(c, acc, e, qj, p, Bt, W1t, b1, W2t, b2, ln3w, ln3b):
    off = c * GC
    full = lambda shape: pl.BlockSpec(shape, lambda i: (0,) * len(shape))
    first = acc is None
    return pl.pallas_call(
        _phase2_body,
        grid=(GC,),
        in_specs=[
            pl.BlockSpec((1, 8, D), lambda i: (0, 0, 0)),
            pl.BlockSpec((BN, K, D), lambda i: (i + off, 0, 0)),
            pl.BlockSpec((BN * K, D // 2), lambda i: (i, 0)),
            pl.BlockSpec((BN, D), lambda i: (i + off, 0)),
            full((D, D)), full((D, D)), full((1, D)), full((D, D)), full((1, D)),
            full((1, D)), full((1, D)),
        ],
        out_specs=pl.BlockSpec((BN, K, D), lambda i: (i + off, 0, 0)),
        out_shape=jax.ShapeDtypeStruct((N, K, D), jnp.float32),
        input_output_aliases=({} if first else {0: 0}),
    )(e if first else acc, e, qj, p,
      Bt, W1t, b1.reshape(1, D), W2t, b2.reshape(1, D),
      ln3w.reshape(1, D), ln3b.reshape(1, D))


def kernel(node_features, edge_features, neighbor_indices, mask,
           em_W0, em_b0, em_W1, em_b1, em_W2, em_b2, ln1_w, ln1_b,
           d_W0, d_b0, d_W1, d_b1, ln2_w, ln2_b,
           eu_W0, eu_b0, eu_W1, eu_b1, eu_W2, eu_b2, ln3_w, ln3_b):
    h = node_features
    e = edge_features
    pad_idx = ((jnp.arange(C * (NE_CALL - NE_C), dtype=jnp.int32) * 131) % N
               ).reshape(C, NE_CALL - NE_C)
    idx_all = jnp.concatenate(
        [neighbor_indices.reshape(C, NE_C), pad_idx], axis=1
    ).reshape(C, NC_CALL, ECHUNK)
    mask2d = mask.reshape(N, 1)

    A1t = em_W0[:, :D].T
    B1t = em_W0[:, D:2 * D].T
    C1t = em_W0[:, 2 * D:].T
    A2t = eu_W0[:, :D].T
    B2t = eu_W0[:, D:2 * D].T
    C2t = eu_W0[:, 2 * D:].T

    pack32 = lambda q: lax.bitcast_convert_type(
        jnp.stack([q[:, :D // 2], q[:, D // 2:]], axis=-1), jnp.int32)

    p1, q1 = _prep(h, A1t, C1t, em_b0)
    q1_32 = pack32(q1)
    qj1 = [_gather(q1_32, idx_all[c]) for c in range(C)]
    parts = [_phase1(c, e, qj1[c], p1, h, mask2d,
                     B1t, em_W1.T, em_b1, em_W2.T, em_b2, ln1_w, ln1_b,
                     d_W0.T, d_b0, d_W1.T, d_b1, ln2_w, ln2_b,
                     A2t, C2t, eu_b0) for c in range(C)]
    h_new = jnp.concatenate([pt[0] for pt in parts])
    p2 = jnp.concatenate([pt[1] for pt in parts])
    q2 = jnp.concatenate([pt[2] for pt in parts])

    q2_32 = pack32(q2)
    acc = None
    for c in range(C):
        qj2_c = _gather(q2_32, idx_all[c])
        acc = _phase2(c, acc, e, qj2_c, p2,
                      B2t, eu_W1.T, eu_b1, eu_W2.T, eu_b2, ln3_w, ln3_b)
    return (h_new, acc)


# speedup vs baseline: 1.2924x; 1.2924x over previous
import functools

import jax
import jax.numpy as jnp
from jax import lax
from jax.experimental import pallas as pl
from jax.experimental.pallas import tpu as pltpu
from jax.experimental.pallas import tpu_sc as plsc

N, K, D, H = 10000, 32, 128, 512
BN = 200
GRID = N // BN
NE = N * K
ECHUNK = 128
NW = 32
C = 2
NE_C = NE // C
NC_CALL = -(-NE_C // ECHUNK // (2 * NW)) * 2 * NW
PW = NC_CALL // NW
NE_CALL = NC_CALL * ECHUNK
GC = GRID // C


def _ln(x, w, b):
    m = jnp.mean(x, axis=-1, keepdims=True)
    v = jnp.mean(jnp.square(x - m), axis=-1, keepdims=True)
    return (x - m) * lax.rsqrt(v + 1e-5) * w + b


def _gelu(x):
    return 0.5 * x * (1.0 + lax.erf(x * 0.7071067811865476))


def _prep_body(h_ref, at_ref, ct_ref, b0_ref, p_ref, q_ref):
    h = h_ref[...]
    p_ref[...] = jnp.dot(h, at_ref[...], preferred_element_type=jnp.float32) + b0_ref[...]
    q_ref[...] = jnp.dot(h, ct_ref[...], preferred_element_type=jnp.float32)


def _prep(h, At, Ct, b0):
    return pl.pallas_call(
        _prep_body,
        out_shape=(
            jax.ShapeDtypeStruct((N, D), jnp.float32),
            jax.ShapeDtypeStruct((N, D), jnp.float32),
        ),
    )(h, At, Ct, b0.reshape(1, D))


def _gather_body(table_hbm, idx_hbm, out_hbm, idx_v, rows0, rows1, sem0, sem1):
    wid = lax.axis_index("s") * 2 + lax.axis_index("c")
    base = wid * PW

    pltpu.sync_copy(idx_hbm.at[pl.ds(base, PW)], idx_v)

    def start(c, rows, sem):
        pltpu.async_copy(table_hbm.at[idx_v.at[c]], rows, sem)

    def wait(rows, sem):
        pltpu.make_async_copy(out_hbm.at[pl.ds(0, ECHUNK)], rows, sem).wait()

    def writeback(c, rows):
        pltpu.sync_copy(rows, out_hbm.at[pl.ds((base + c) * ECHUNK, ECHUNK)])

    start(0, rows0, sem0)

    def body(t, carry):
        c0 = 2 * t
        start(c0 + 1, rows1, sem1)
        wait(rows0, sem0)
        writeback(c0, rows0)

        @pl.when(c0 + 2 < PW)
        def _():
            start(c0 + 2, rows0, sem0)

        wait(rows1, sem1)
        writeback(c0 + 1, rows1)
        return carry

    lax.fori_loop(0, PW // 2, body, 0)


@functools.lru_cache(maxsize=None)
def _make_gather():
    return pl.kernel(
        _gather_body,
        out_type=jax.ShapeDtypeStruct((NE_CALL, D), jnp.float32),
        mesh=plsc.VectorSubcoreMesh(core_axis_name="c", subcore_axis_name="s"),
        scratch_types=[
            pltpu.VMEM((PW, ECHUNK), jnp.int32),
            pltpu.VMEM((ECHUNK, D), jnp.float32),
            pltpu.VMEM((ECHUNK, D), jnp.float32),
            pltpu.SemaphoreType.DMA,
            pltpu.SemaphoreType.DMA,
        ],
    )


def _gather(table, idx):
    return _make_gather()(table, idx)


def _phase1_body(e_ref, qj_ref, p_ref, h_ref, mask_ref,
                 bt_ref, w1t_ref, b1_ref, w2t_ref, b2_ref,
                 ln1w_ref, ln1b_ref,
                 dw0t_ref, db0_ref, dw1t_ref, db1_ref,
                 ln2w_ref, ln2b_ref, a2t_ref, c2t_ref, eub0_ref,
                 out_ref, p2_ref, q2_ref):
    e = e_ref[...].reshape(BN * K, D)
    x = jnp.dot(e, bt_ref[...], preferred_element_type=jnp.float32) + qj_ref[...]
    p = jnp.broadcast_to(p_ref[...][:, None, :], (BN, K, D)).reshape(BN * K, D)
    x = _gelu(x + p)
    x = _gelu(jnp.dot(x, w1t_ref[...], preferred_element_type=jnp.float32) + b1_ref[...])
    m = jnp.dot(x, w2t_ref[...], preferred_element_type=jnp.float32) + b2_ref[...]
    agg = jnp.sum(m.reshape(BN, K, D), axis=1) * (1.0 / 30.0)
    h1 = _ln(h_ref[...] + agg, ln1w_ref[...], ln1b_ref[...])
    d = _gelu(jnp.dot(h1, dw0t_ref[...], preferred_element_type=jnp.float32) + db0_ref[...])
    h2 = h1 + jnp.dot(d, dw1t_ref[...], preferred_element_type=jnp.float32) + db1_ref[...]
    h2 = _ln(h2, ln2w_ref[...], ln2b_ref[...])
    h2 = h2 * mask_ref[...]
    out_ref[...] = h2
    p2_ref[...] = jnp.dot(h2, a2t_ref[...], preferred_element_type=jnp.float32) + eub0_ref[...]
    q2_ref[...] = jnp.dot(h2, c2t_ref[...], preferred_element_type=jnp.float32)


def _phase1(c, e, qj, p, h, mask2d, Bt, W1t, b1, W2t, b2, ln1w, ln1b,
            dW0t, db0, dW1t, db1, ln2w, ln2b, A2t, C2t, eu_b0):
    off = c * GC
    full = lambda shape: pl.BlockSpec(shape, lambda i: (0,) * len(shape))
    nd_off = pl.BlockSpec((BN, D), lambda i: (i + off, 0))
    nd_loc = pl.BlockSpec((BN, D), lambda i: (i, 0))
    return pl.pallas_call(
        _phase1_body,
        grid=(GC,),
        in_specs=[
            pl.BlockSpec((BN, K, D), lambda i: (i + off, 0, 0)),
            pl.BlockSpec((BN * K, D), lambda i: (i, 0)),
            nd_off,
            nd_off,
            pl.BlockSpec((BN, 1), lambda i: (i + off, 0)),
            full((D, D)), full((D, D)), full((1, D)), full((D, D)), full((1, D)),
            full((1, D)), full((1, D)),
            full((D, H)), full((1, H)), full((H, D)), full((1, D)),
            full((1, D)), full((1, D)),
            full((D, D)), full((D, D)), full((1, D)),
        ],
        out_specs=(nd_loc, nd_loc, nd_loc),
        out_shape=(
            jax.ShapeDtypeStruct((N // C, D), jnp.float32),
            jax.ShapeDtypeStruct((N // C, D), jnp.float32),
            jax.ShapeDtypeStruct((N // C, D), jnp.float32),
        ),
    )(e, qj, p, h, mask2d, Bt, W1t, b1.reshape(1, D), W2t, b2.reshape(1, D),
      ln1w.reshape(1, D), ln1b.reshape(1, D),
      dW0t, db0.reshape(1, H), dW1t, db1.reshape(1, D),
      ln2w.reshape(1, D), ln2b.reshape(1, D), A2t, C2t, eu_b0.reshape(1, D))


def _phase2_body(acc_ref, e_ref, qj_ref, p_ref,
                 bt_ref, w1t_ref, b1_ref, w2t_ref, b2_ref,
                 ln3w_ref, ln3b_ref, out_ref):
    del acc_ref
    e = e_ref[...].reshape(BN * K, D)
    x = jnp.dot(e, bt_ref[...], preferred_element_type=jnp.float32) + qj_ref[...]
    p = jnp.broadcast_to(p_ref[...][:, None, :], (BN, K, D)).reshape(BN * K, D)
    x = _gelu(x + p)
    x = _gelu(jnp.dot(x, w1t_ref[...], preferred_element_type=jnp.float32) + b1_ref[...])
    m = jnp.dot(x, w2t_ref[...], preferred_element_type=jnp.float32) + b2_ref[...]
    out = _ln(e + m, ln3w_ref[...], ln3b_ref[...])
    out_ref[...] = out.reshape(BN, K, D)


def _phase2(c, acc, e, qj, p, Bt, W1t, b1, W2t, b2, ln3w, ln3b):
    off = c * GC
    full = lambda shape: pl.BlockSpec(shape, lambda i: (0,) * len(shape))
    first = acc is None
    return pl.pallas_call(
        _phase2_body,
        grid=(GC,),
        in_specs=[
            pl.BlockSpec((1, 8, D), lambda i: (0, 0, 0)),
            pl.BlockSpec((BN, K, D), lambda i: (i + off, 0, 0)),
            pl.BlockSpec((BN * K, D), lambda i: (i, 0)),
            pl.BlockSpec((BN, D), lambda i: (i + off, 0)),
            full((D, D)), full((D, D)), full((1, D)), full((D, D)), full((1, D)),
            full((1, D)), full((1, D)),
        ],
        out_specs=pl.BlockSpec((BN, K, D), lambda i: (i + off, 0, 0)),
        out_shape=jax.ShapeDtypeStruct((N, K, D), jnp.float32),
        input_output_aliases=({} if first else {0: 0}),
    )(e if first else acc, e, qj, p,
      Bt, W1t, b1.reshape(1, D), W2t, b2.reshape(1, D),
      ln3w.reshape(1, D), ln3b.reshape(1, D))


def kernel(node_features, edge_features, neighbor_indices, mask,
           em_W0, em_b0, em_W1, em_b1, em_W2, em_b2, ln1_w, ln1_b,
           d_W0, d_b0, d_W1, d_b1, ln2_w, ln2_b,
           eu_W0, eu_b0, eu_W1, eu_b1, eu_W2, eu_b2, ln3_w, ln3_b):
    h = node_features
    e = edge_features
    pad_idx = ((jnp.arange(C * (NE_CALL - NE_C), dtype=jnp.int32) * 131) % N
               ).reshape(C, NE_CALL - NE_C)
    idx_all = jnp.concatenate(
        [neighbor_indices.reshape(C, NE_C), pad_idx], axis=1
    ).reshape(C, NC_CALL, ECHUNK)
    mask2d = mask.reshape(N, 1)

    A1t = em_W0[:, :D].T
    B1t = em_W0[:, D:2 * D].T
    C1t = em_W0[:, 2 * D:].T
    A2t = eu_W0[:, :D].T
    B2t = eu_W0[:, D:2 * D].T
    C2t = eu_W0[:, 2 * D:].T

    p1, q1 = _prep(h, A1t, C1t, em_b0)
    qj1 = [_gather(q1, idx_all[c]) for c in range(C)]
    parts = [_phase1(c, e, qj1[c], p1, h, mask2d,
                     B1t, em_W1.T, em_b1, em_W2.T, em_b2, ln1_w, ln1_b,
                     d_W0.T, d_b0, d_W1.T, d_b1, ln2_w, ln2_b,
                     A2t, C2t, eu_b0) for c in range(C)]
    h_new = jnp.concatenate([pt[0] for pt in parts])
    p2 = jnp.concatenate([pt[1] for pt in parts])
    q2 = jnp.concatenate([pt[2] for pt in parts])

    acc = None
    for c in range(C):
        qj2_c = _gather(q2, idx_all[c])
        acc = _phase2(c, acc, e, qj2_c, p2,
                      B2t, eu_W1.T, eu_b1, eu_W2.T, eu_b2, ln3_w, ln3_b)
    return (h_new, acc)


# speedup vs baseline: 1.3069x; 1.0112x over previous
import functools

import jax
import jax.numpy as jnp
from jax import lax
from jax.experimental import pallas as pl
from jax.experimental.pallas import tpu as pltpu
from jax.experimental.pallas import tpu_sc as plsc

N, K, D, H = 10000, 32, 128, 512
BN = 200
GRID = N // BN
NE = N * K
ECHUNK = 128
NW = 32
C = 5
NE_C = NE // C
NC_CALL = -(-NE_C // ECHUNK // (2 * NW)) * 2 * NW
PW = NC_CALL // NW
NE_CALL = NC_CALL * ECHUNK
GC = GRID // C


def _ln(x, w, b):
    m = jnp.mean(x, axis=-1, keepdims=True)
    v = jnp.mean(jnp.square(x - m), axis=-1, keepdims=True)
    return (x - m) * lax.rsqrt(v + 1e-5) * w + b


def _gelu(x):
    return 0.5 * x * (1.0 + lax.erf(x * 0.7071067811865476))


def _prep_body(h_ref, at_ref, ct_ref, b0_ref, p_ref, q_ref):
    h = h_ref[...]
    p_ref[...] = jnp.dot(h, at_ref[...], preferred_element_type=jnp.float32) + b0_ref[...]
    q_ref[...] = jnp.dot(h, ct_ref[...], preferred_element_type=jnp.float32)


def _prep(h, At, Ct, b0):
    return pl.pallas_call(
        _prep_body,
        out_shape=(
            jax.ShapeDtypeStruct((N, D), jnp.float32),
            jax.ShapeDtypeStruct((N, D), jnp.float32),
        ),
    )(h, At, Ct, b0.reshape(1, D))


def _gather_body(table_hbm, idx_hbm, out_hbm, idx_v, rows0, rows1, sem0, sem1):
    wid = lax.axis_index("s") * 2 + lax.axis_index("c")
    base = wid * PW

    pltpu.sync_copy(idx_hbm.at[pl.ds(base, PW)], idx_v)

    def start(c, rows, sem):
        pltpu.async_copy(table_hbm.at[idx_v.at[c]], rows, sem)

    def wait(rows, sem):
        pltpu.make_async_copy(out_hbm.at[pl.ds(0, ECHUNK)], rows, sem).wait()

    def writeback(c, rows):
        pltpu.sync_copy(rows, out_hbm.at[pl.ds((base + c) * ECHUNK, ECHUNK)])

    start(0, rows0, sem0)

    def body(t, carry):
        c0 = 2 * t
        start(c0 + 1, rows1, sem1)
        wait(rows0, sem0)
        writeback(c0, rows0)

        @pl.when(c0 + 2 < PW)
        def _():
            start(c0 + 2, rows0, sem0)

        wait(rows1, sem1)
        writeback(c0 + 1, rows1)
        return carry

    lax.fori_loop(0, PW // 2, body, 0)


@functools.lru_cache(maxsize=None)
def _make_gather():
    return pl.kernel(
        _gather_body,
        out_type=jax.ShapeDtypeStruct((NE_CALL, D), jnp.float32),
        mesh=plsc.VectorSubcoreMesh(core_axis_name="c", subcore_axis_name="s"),
        scratch_types=[
            pltpu.VMEM((PW, ECHUNK), jnp.int32),
            pltpu.VMEM((ECHUNK, D), jnp.float32),
            pltpu.VMEM((ECHUNK, D), jnp.float32),
            pltpu.SemaphoreType.DMA,
            pltpu.SemaphoreType.DMA,
        ],
    )


def _gather(table, idx):
    return _make_gather()(table, idx)


def _phase1_body(e_ref, qj_ref, p_ref, h_ref, mask_ref,
                 bt_ref, w1t_ref, b1_ref, w2t_ref, b2_ref,
                 ln1w_ref, ln1b_ref,
                 dw0t_ref, db0_ref, dw1t_ref, db1_ref,
                 ln2w_ref, ln2b_ref, a2t_ref, c2t_ref, eub0_ref,
                 out_ref, p2_ref, q2_ref):
    e = e_ref[...].reshape(BN * K, D)
    x = jnp.dot(e, bt_ref[...], preferred_element_type=jnp.float32) + qj_ref[...]
    p = jnp.broadcast_to(p_ref[...][:, None, :], (BN, K, D)).reshape(BN * K, D)
    x = _gelu(x + p)
    x = _gelu(jnp.dot(x, w1t_ref[...], preferred_element_type=jnp.float32) + b1_ref[...])
    m = jnp.dot(x, w2t_ref[...], preferred_element_type=jnp.float32) + b2_ref[...]
    agg = jnp.sum(m.reshape(BN, K, D), axis=1) * (1.0 / 30.0)
    h1 = _ln(h_ref[...] + agg, ln1w_ref[...], ln1b_ref[...])
    d = _gelu(jnp.dot(h1, dw0t_ref[...], preferred_element_type=jnp.float32) + db0_ref[...])
    h2 = h1 + jnp.dot(d, dw1t_ref[...], preferred_element_type=jnp.float32) + db1_ref[...]
    h2 = _ln(h2, ln2w_ref[...], ln2b_ref[...])
    h2 = h2 * mask_ref[...]
    out_ref[...] = h2
    p2_ref[...] = jnp.dot(h2, a2t_ref[...], preferred_element_type=jnp.float32) + eub0_ref[...]
    q2_ref[...] = jnp.dot(h2, c2t_ref[...], preferred_element_type=jnp.float32)


def _phase1(c, e, qj, p, h, mask2d, Bt, W1t, b1, W2t, b2, ln1w, ln1b,
            dW0t, db0, dW1t, db1, ln2w, ln2b, A2t, C2t, eu_b0):
    off = c * GC
    full = lambda shape: pl.BlockSpec(shape, lambda i: (0,) * len(shape))
    nd_off = pl.BlockSpec((BN, D), lambda i: (i + off, 0))
    nd_loc = pl.BlockSpec((BN, D), lambda i: (i, 0))
    return pl.pallas_call(
        _phase1_body,
        grid=(GC,),
        in_specs=[
            pl.BlockSpec((BN, K, D), lambda i: (i + off, 0, 0)),
            pl.BlockSpec((BN * K, D), lambda i: (i, 0)),
            nd_off,
            nd_off,
            pl.BlockSpec((BN, 1), lambda i: (i + off, 0)),
            full((D, D)), full((D, D)), full((1, D)), full((D, D)), full((1, D)),
            full((1, D)), full((1, D)),
            full((D, H)), full((1, H)), full((H, D)), full((1, D)),
            full((1, D)), full((1, D)),
            full((D, D)), full((D, D)), full((1, D)),
        ],
        out_specs=(nd_loc, nd_loc, nd_loc),
        out_shape=(
            jax.ShapeDtypeStruct((N // C, D), jnp.float32),
            jax.ShapeDtypeStruct((N // C, D), jnp.float32),
            jax.ShapeDtypeStruct((N // C, D), jnp.float32),
        ),
    )(e, qj, p, h, mask2d, Bt, W1t, b1.reshape(1, D), W2t, b2.reshape(1, D),
      ln1w.reshape(1, D), ln1b.reshape(1, D),
      dW0t, db0.reshape(1, H), dW1t, db1.reshape(1, D),
      ln2w.reshape(1, D), ln2b.reshape(1, D), A2t, C2t, eu_b0.reshape(1, D))


def _phase2_body(acc_ref, e_ref, qj_ref, p_ref,
                 bt_ref, w1t_ref, b1_ref, w2t_ref, b2_ref,
                 ln3w_ref, ln3b_ref, out_ref):
    del acc_ref
    e = e_ref[...].reshape(BN * K, D)
    x = jnp.dot(e, bt_ref[...], preferred_element_type=jnp.float32) + qj_ref[...]
    p = jnp.broadcast_to(p_ref[...][:, None, :], (BN, K, D)).reshape(BN * K, D)
    x = _gelu(x + p)
    x = _gelu(jnp.dot(x, w1t_ref[...], preferred_element_type=jnp.float32) + b1_ref[...])
    m = jnp.dot(x, w2t_ref[...], preferred_element_type=jnp.float32) + b2_ref[...]
    out = _ln(e + m, ln3w_ref[...], ln3b_ref[...])
    out_ref[...] = out.reshape(BN, K, D)


def _phase2(c, acc, e, qj, p, Bt, W1t, b1, W2t, b2, ln3w, ln3b):
    off = c * GC
    full = lambda shape: pl.BlockSpec(shape, lambda i: (0,) * len(shape))
    first = acc is None
    return pl.pallas_call(
        _phase2_body,
        grid=(GC,),
        in_specs=[
            pl.BlockSpec((1, 8, D), lambda i: (0, 0, 0)),
            pl.BlockSpec((BN, K, D), lambda i: (i + off, 0, 0)),
            pl.BlockSpec((BN * K, D), lambda i: (i, 0)),
            pl.BlockSpec((BN, D), lambda i: (i + off, 0)),
            full((D, D)), full((D, D)), full((1, D)), full((D, D)), full((1, D)),
            full((1, D)), full((1, D)),
        ],
        out_specs=pl.BlockSpec((BN, K, D), lambda i: (i + off, 0, 0)),
        out_shape=jax.ShapeDtypeStruct((N, K, D), jnp.float32),
        input_output_aliases=({} if first else {0: 0}),
    )(e if first else acc, e, qj, p,
      Bt, W1t, b1.reshape(1, D), W2t, b2.reshape(1, D),
      ln3w.reshape(1, D), ln3b.reshape(1, D))


def kernel(node_features, edge_features, neighbor_indices, mask,
           em_W0, em_b0, em_W1, em_b1, em_W2, em_b2, ln1_w, ln1_b,
           d_W0, d_b0, d_W1, d_b1, ln2_w, ln2_b,
           eu_W0, eu_b0, eu_W1, eu_b1, eu_W2, eu_b2, ln3_w, ln3_b):
    h = node_features
    e = edge_features
    pad_idx = ((jnp.arange(C * (NE_CALL - NE_C), dtype=jnp.int32) * 131) % N
               ).reshape(C, NE_CALL - NE_C)
    idx_all = jnp.concatenate(
        [neighbor_indices.reshape(C, NE_C), pad_idx], axis=1
    ).reshape(C, NC_CALL, ECHUNK)
    mask2d = mask.reshape(N, 1)

    A1t = em_W0[:, :D].T
    B1t = em_W0[:, D:2 * D].T
    C1t = em_W0[:, 2 * D:].T
    A2t = eu_W0[:, :D].T
    B2t = eu_W0[:, D:2 * D].T
    C2t = eu_W0[:, 2 * D:].T

    p1, q1 = _prep(h, A1t, C1t, em_b0)
    qj1 = [_gather(q1, idx_all[c]) for c in range(C)]
    parts = [_phase1(c, e, qj1[c], p1, h, mask2d,
                     B1t, em_W1.T, em_b1, em_W2.T, em_b2, ln1_w, ln1_b,
                     d_W0.T, d_b0, d_W1.T, d_b1, ln2_w, ln2_b,
                     A2t, C2t, eu_b0) for c in range(C)]
    h_new = jnp.concatenate([pt[0] for pt in parts])
    p2 = jnp.concatenate([pt[1] for pt in parts])
    q2 = jnp.concatenate([pt[2] for pt in parts])

    acc = None
    for c in range(C):
        qj2_c = _gather(q2, idx_all[c])
        acc = _phase2(c, acc, e, qj2_c, p2,
                      B2t, eu_W1.T, eu_b1, eu_W2.T, eu_b2, ln3_w, ln3_b)
    return (h_new, acc)
